# baseline (speedup 1.0000x reference)
import functools

import jax
import jax.numpy as jnp
from jax import lax
from jax.experimental import pallas as pl
from jax.experimental.pallas import tpu as pltpu
from jax.experimental.pallas import tpu_sc as plsc

N = 50000
E = 800000
L = 100000
H = 64
HH = 32
F = 128

NT = 16
CH = 128

GB = 2
GR = GB * CH
IDEP = 4

EPAD = 802816
EPT = EPAD // NT
NCHE = EPT // CH
NGRP = NCHE // GB

NP = 53248
RPT = NP // NT

DGB = 7
DGR = DGB * CH
LPAD = 114688
LPT = LPAD // NT
NCHL = LPT // CH
LGRP = NCHL // DGB

@functools.cache
def _mesh():
    return plsc.VectorSubcoreMesh(core_axis_name="c", subcore_axis_name="s",
                                  num_cores=2, num_subcores=NT)


def _sc_counts(seg2):
    @functools.partial(
        pl.kernel,
        out_type=jax.ShapeDtypeStruct((2, NP, 16), jnp.float32),
        mesh=_mesh(),
        compiler_params=pltpu.CompilerParams(use_tc_tiling_on_sc=False),
        scratch_types=[
            pltpu.VMEM_SHARED((NP, 16), jnp.float32),
            pltpu.VMEM((NCHE, CH), jnp.int32),
            pltpu.VMEM((CH, 16), jnp.float32),
            pltpu.VMEM((GR, 16), jnp.float32),
            pltpu.SemaphoreType.DMA,
        ],
    )
    def k(seg_h, out_h, acc, svb, ones_v, zbuf, ssem):
        c = lax.axis_index("c")
        s = lax.axis_index("s")
        z16 = jnp.zeros((16,), jnp.float32)
        o16 = jnp.ones((16,), jnp.float32)

        def fill(i, _):
            zbuf[i, :] = z16
            return _
        lax.fori_loop(0, GR, fill, None)

        def fill_o(i, _):
            ones_v[i, :] = o16
            return _
        lax.fori_loop(0, CH, fill_o, None)

        r0 = s * RPT

        def zero_acc(i, _):
            pltpu.sync_copy(zbuf, acc.at[pl.ds(r0 + i * GR, GR)])
            return _
        lax.fori_loop(0, RPT // GR, zero_acc, None)
        pltpu.sync_copy(seg_h.at[c].at[s], svb)
        plsc.subcore_barrier()

        ngr = NCHE // GB
        for b in range(GB):
            pltpu.async_copy(ones_v, acc.at[svb.at[b]], ssem, add=True)

        def body(g, _):
            for b in range(GB):
                pltpu.async_copy(ones_v, acc.at[svb.at[g * GB + b]], ssem,
                                 add=True)
            for b in range(GB):
                pltpu.make_async_copy(
                    ones_v, acc.at[svb.at[(g - 1) * GB + b]], ssem).wait()
            return _
        lax.fori_loop(1, ngr, body, None)
        for b in range(GB):
            pltpu.make_async_copy(
                ones_v, acc.at[svb.at[(ngr - 1) * GB + b]], ssem).wait()
        plsc.subcore_barrier()
        pltpu.sync_copy(acc.at[pl.ds(r0, RPT)], out_h.at[c].at[pl.ds(r0, RPT)])

    return k(seg2)


def _sc_agg(tab2, gidx2, sidx):
    @functools.partial(
        pl.kernel,
        out_type=jax.ShapeDtypeStruct((2, NP, HH), jnp.float32),
        mesh=_mesh(),
        compiler_params=pltpu.CompilerParams(use_tc_tiling_on_sc=False),
        scratch_types=[
            pltpu.VMEM_SHARED((NP, HH), jnp.float32),
            pltpu.VMEM((IDEP, GB, CH), jnp.int32),
            pltpu.VMEM((IDEP, GB, CH), jnp.int32),
            pltpu.VMEM((2, GR, HH), jnp.float32),
            pltpu.SemaphoreType.DMA,
            pltpu.SemaphoreType.DMA,
            pltpu.SemaphoreType.DMA,
        ],
    )
    def k(tab_h, g_h, s_h, out_h, acc, gvb, svb, rows, isem, gsem, ssem):
        c = lax.axis_index("c")
        s = lax.axis_index("s")
        z16 = jnp.zeros((16,), jnp.float32)

        def fill_z(i, _):
            rows[0, i, pl.ds(0, 16)] = z16
            rows[0, i, pl.ds(16, 16)] = z16
            return _
        lax.fori_loop(0, GR, fill_z, None)

        r0 = s * RPT

        def zero_acc(i, _):
            pltpu.sync_copy(rows.at[0], acc.at[pl.ds(r0 + i * GR, GR)])
            return _
        lax.fori_loop(0, RPT // GR, zero_acc, None)

        def idx_load(g, q):
            pltpu.async_copy(g_h.at[c].at[s].at[pl.ds(g * GB, GB)],
                             gvb.at[q], isem)
            pltpu.async_copy(s_h.at[s].at[pl.ds(g * GB, GB)],
                             svb.at[q], isem)

        def idx_drain(g, q):
            pltpu.make_async_copy(g_h.at[c].at[s].at[pl.ds(g * GB, GB)],
                                  gvb.at[q], isem).wait()
            pltpu.make_async_copy(s_h.at[s].at[pl.ds(g * GB, GB)],
                                  svb.at[q], isem).wait()

        def gath(q, p):
            for b in range(GB):
                pltpu.async_copy(tab_h.at[gvb.at[q].at[b]],
                                 rows.at[p].at[pl.ds(b * CH, CH)], gsem)

        def gath_drain(q, p):
            for b in range(GB):
                pltpu.make_async_copy(tab_h.at[gvb.at[q].at[b]],
                                      rows.at[p].at[pl.ds(b * CH, CH)],
                                      gsem).wait()

        def scat(q, p):
            for b in range(GB):
                pltpu.async_copy(rows.at[p].at[pl.ds(b * CH, CH)],
                                 acc.at[svb.at[q].at[b]], ssem, add=True)

        def scat_drain(q, p):
            for b in range(GB):
                pltpu.make_async_copy(rows.at[p].at[pl.ds(b * CH, CH)],
                                      acc.at[svb.at[q].at[b]], ssem).wait()

        idx_load(0, 0)
        idx_load(1, 1)
        idx_drain(0, 0)
        gath(0, 0)
        plsc.subcore_barrier()

        def body(g, _):
            p = g % 2
            q = g % IDEP

            @pl.when(g >= 2)
            def _free_rows():
                scat_drain((g - 2) % IDEP, p)

            @pl.when(g + 1 < NGRP)
            def _prefetch():
                idx_load(g + 1, (g + 1) % IDEP)

            idx_drain(g, q)
            gath(q, p)
            gath_drain((g - 1) % IDEP, 1 - p)
            scat((g - 1) % IDEP, 1 - p)
            return _
        lax.fori_loop(1, NGRP, body, None)

        last = NGRP - 1
        lp = last % 2
        scat_drain((last - 1) % IDEP, 1 - lp)
        gath_drain(last % IDEP, lp)
        scat(last % IDEP, lp)
        scat_drain(last % IDEP, lp)

        plsc.subcore_barrier()
        pltpu.sync_copy(acc.at[pl.ds(r0, RPT)], out_h.at[c].at[pl.ds(r0, RPT)])

    return k(tab2, gidx2, sidx)


def _sc_dec_gather(uf, mf, eli2):
    @functools.partial(
        pl.kernel,
        out_type=jax.ShapeDtypeStruct((2, LPAD, H), jnp.float32),
        mesh=_mesh(),
        compiler_params=pltpu.CompilerParams(use_tc_tiling_on_sc=False),
        scratch_types=[
            pltpu.VMEM((NCHL, CH), jnp.int32),
            pltpu.VMEM((2, DGR, H), jnp.float32),
            pltpu.SemaphoreType.DMA,
            pltpu.SemaphoreType.DMA,
        ],
    )
    def k(uf_h, mf_h, e_h, out_h, evb, rows, gsem, osem):
        c = lax.axis_index("c")
        s = lax.axis_index("s")
        pltpu.sync_copy(e_h.at[c].at[s], evb)

        def gath(tab_h, g, p):
            for b in range(DGB):
                pltpu.async_copy(tab_h.at[evb.at[g * DGB + b]],
                                 rows.at[p].at[pl.ds(b * CH, CH)], gsem)

        def gath_drain(tab_h, g, p):
            for b in range(DGB):
                pltpu.make_async_copy(tab_h.at[evb.at[g * DGB + b]],
                                      rows.at[p].at[pl.ds(b * CH, CH)],
                                      gsem).wait()

        def out_ref(g):
            return out_h.at[c].at[pl.ds(s * LPT + g * DGR, DGR)]

        def run(tab_h):
            gath(tab_h, 0, 0)

            def body(g, _):
                p = g % 2
                @pl.when(g >= 2)
                def _drain():
                    pltpu.make_async_copy(rows.at[p], out_ref(g - 2),
                                          osem).wait()
                gath(tab_h, g, p)
                gath_drain(tab_h, g - 1, 1 - p)
                pltpu.async_copy(rows.at[1 - p], out_ref(g - 1), osem)
                return _
            lax.fori_loop(1, LGRP, body, None)
            last = LGRP - 1
            lp = last % 2
            pltpu.make_async_copy(rows.at[1 - lp], out_ref(last - 1),
                                  osem).wait()
            gath_drain(tab_h, last, lp)
            pltpu.sync_copy(rows.at[lp], out_ref(last))

        @pl.when(c == 0)
        def _u():
            run(uf_h)

        @pl.when(c == 1)
        def _m():
            run(mf_h)

    return k(uf, mf, eli2)


_BN = 2000


def _tc_prep(movie_x, user_emb, wp, bp):
    def body(mx_ref, ue_ref, wp_ref, bp_ref, om_ref, ou_ref):
        m = jnp.maximum(
            jnp.dot(mx_ref[...], wp_ref[...],
                    preferred_element_type=jnp.float32) + bp_ref[...], 0.0)
        om_ref[0] = m[:, :HH]
        om_ref[1] = m[:, HH:]
        ue = ue_ref[...]
        ou_ref[0] = ue[:, :HH]
        ou_ref[1] = ue[:, HH:]

    grid = (N // _BN,)
    return pl.pallas_call(
        body,
        grid=grid,
        in_specs=[
            pl.BlockSpec((_BN, F), lambda i: (i, 0)),
            pl.BlockSpec((_BN, H), lambda i: (i, 0)),
            pl.BlockSpec((F, H), lambda i: (0, 0)),
            pl.BlockSpec((1, H), lambda i: (0, 0)),
        ],
        out_specs=[
            pl.BlockSpec((2, _BN, HH), lambda i: (0, i, 0)),
            pl.BlockSpec((2, _BN, HH), lambda i: (0, i, 0)),
        ],
        out_shape=[
            jax.ShapeDtypeStruct((2, N, HH), jnp.float32),
            jax.ShapeDtypeStruct((2, N, HH), jnp.float32),
        ],
    )(movie_x, user_emb, wp, bp.reshape(1, H))


def _tc_combine1(s_agg, cnts, x_p, wl, bl, wr, which):
    def body(s_ref, c_ref, x_ref, wl_ref, bl_ref, wr_ref, o_ref):
        cnt = jnp.maximum(c_ref[0, :, :1], 1.0)
        wl_ = wl_ref[...]
        wr_ = wr_ref[...]
        a_lo = s_ref[0] / cnt
        a_hi = s_ref[1] / cnt
        z = (jnp.dot(a_lo, wl_[:HH], preferred_element_type=jnp.float32)
             + jnp.dot(a_hi, wl_[HH:], preferred_element_type=jnp.float32)
             + jnp.dot(x_ref[0], wr_[:HH], preferred_element_type=jnp.float32)
             + jnp.dot(x_ref[1], wr_[HH:], preferred_element_type=jnp.float32)
             + bl_ref[...])
        x_full = jnp.concatenate([x_ref[0], x_ref[1]], axis=1)
        r = x_full + jnp.maximum(z, 0.0)
        o_ref[0] = r[:, :HH]
        o_ref[1] = r[:, HH:]

    return pl.pallas_call(
        body,
        grid=(N // _BN,),
        in_specs=[
            pl.BlockSpec((2, _BN, HH), lambda i: (0, i, 0)),
            pl.BlockSpec((1, _BN, 16), lambda i: (which, i, 0)),
            pl.BlockSpec((2, _BN, HH), lambda i: (0, i, 0)),
            pl.BlockSpec((H, H), lambda i: (0, 0)),
            pl.BlockSpec((1, H), lambda i: (0, 0)),
            pl.BlockSpec((H, H), lambda i: (0, 0)),
        ],
        out_specs=pl.BlockSpec((2, _BN, HH), lambda i: (0, i, 0)),
        out_shape=jax.ShapeDtypeStruct((2, N, HH), jnp.float32),
    )(s_agg, cnts, x_p, wl, bl.reshape(1, H), wr)


def _tc_combine2(s_agg, cnts, x_p, wl, bl, wr, which):
    def body(s_ref, c_ref, x_ref, wl_ref, bl_ref, wr_ref, o_ref):
        cnt = jnp.maximum(c_ref[0, :, :1], 1.0)
        wl_ = wl_ref[...]
        wr_ = wr_ref[...]
        a_lo = s_ref[0] / cnt
        a_hi = s_ref[1] / cnt
        z = (jnp.dot(a_lo, wl_[:HH], preferred_element_type=jnp.float32)
             + jnp.dot(a_hi, wl_[HH:], preferred_element_type=jnp.float32)
             + jnp.dot(x_ref[0], wr_[:HH], preferred_element_type=jnp.float32)
             + jnp.dot(x_ref[1], wr_[HH:], preferred_element_type=jnp.float32)
             + bl_ref[...])
        nrm = jnp.sqrt(jnp.sum(z * z, axis=1, keepdims=True))
        o_ref[...] = z / jnp.maximum(nrm, 1e-12)

    return pl.pallas_call(
        body,
        grid=(N // _BN,),
        in_specs=[
            pl.BlockSpec((2, _BN, HH), lambda i: (0, i, 0)),
            pl.BlockSpec((1, _BN, 16), lambda i: (which, i, 0)),
            pl.BlockSpec((2, _BN, HH), lambda i: (0, i, 0)),
            pl.BlockSpec((H, H), lambda i: (0, 0)),
            pl.BlockSpec((1, H), lambda i: (0, 0)),
            pl.BlockSpec((H, H), lambda i: (0, 0)),
        ],
        out_specs=pl.BlockSpec((_BN, H), lambda i: (i, 0)),
        out_shape=jax.ShapeDtypeStruct((N, H), jnp.float32),
    )(s_agg, cnts, x_p, wl, bl.reshape(1, H), wr)


_BL = 2000


def _tc_decoder(g, wd1, bd1):
    def body(g_ref, w1_ref, b1_ref, o_ref):
        w1 = w1_ref[...]
        h = (jnp.dot(g_ref[0], w1[:H], preferred_element_type=jnp.float32)
             + jnp.dot(g_ref[1], w1[H:], preferred_element_type=jnp.float32)
             + b1_ref[...])
        o_ref[...] = jnp.maximum(h, 0.0)

    return pl.pallas_call(
        body,
        grid=(L // _BL,),
        in_specs=[
            pl.BlockSpec((2, _BL, H), lambda i: (0, i, 0)),
            pl.BlockSpec((2 * H, H), lambda i: (0, 0)),
            pl.BlockSpec((1, H), lambda i: (0, 0)),
        ],
        out_specs=pl.BlockSpec((_BL, H), lambda i: (i, 0)),
        out_shape=jax.ShapeDtypeStruct((L, H), jnp.float32),
    )(g, wd1, bd1.reshape(1, H))


def kernel(movie_x, edge_index, edge_label_index, user_emb, Wp, bp,
           W1m_l, b1m, W1m_r, W1u_l, b1u, W1u_r,
           W2m_l, b2m, W2m_r, W2u_l, b2u, W2u_r,
           Wd1, bd1, Wd2, bd2):
    i32 = jnp.int32
    src = edge_index[0].astype(i32)
    dst = edge_index[1].astype(i32)
    npad = EPAD - E
    pad0 = jnp.zeros((npad,), i32)
    padn = jnp.full((npad,), N, i32)
    src_g = jnp.concatenate([src, pad0])
    dst_g = jnp.concatenate([dst, pad0])
    src_s = jnp.concatenate([src, padn]).reshape(NT, NCHE, CH)
    dst_s = jnp.concatenate([dst, padn]).reshape(NT, NCHE, CH)
    srcg2 = jnp.concatenate([src_g, src_g + N]).reshape(2, NT, NCHE, CH)
    dstg2 = jnp.concatenate([dst_g, dst_g + N]).reshape(2, NT, NCHE, CH)
    seg2 = jnp.stack([dst_s, src_s])

    cnts = _sc_counts(seg2)
    movie0_p, user0_p = _tc_prep(movie_x, user_emb, Wp, bp)

    s1m = _sc_agg(user0_p.reshape(2 * N, HH), srcg2, dst_s)
    s1u = _sc_agg(movie0_p.reshape(2 * N, HH), dstg2, src_s)
    mr_p = _tc_combine1(s1m, cnts, movie0_p, W1m_l, b1m, W1m_r, which=0)
    ur_p = _tc_combine1(s1u, cnts, user0_p, W1u_l, b1u, W1u_r, which=1)

    s2m = _sc_agg(ur_p.reshape(2 * N, HH), srcg2, dst_s)
    s2u = _sc_agg(mr_p.reshape(2 * N, HH), dstg2, src_s)
    mf = _tc_combine2(s2m, cnts, mr_p, W2m_l, b2m, W2m_r, which=0)
    uf = _tc_combine2(s2u, cnts, ur_p, W2u_l, b2u, W2u_r, which=1)

    eli = edge_label_index.astype(i32)
    lpad0 = jnp.zeros((LPAD - L,), i32)
    e0 = jnp.concatenate([eli[0], lpad0])
    e1 = jnp.concatenate([eli[1], lpad0])
    eli2 = jnp.concatenate([e0, e1]).reshape(2, NT, NCHL, CH)

    g = _sc_dec_gather(uf, mf, eli2)
    h = _tc_decoder(g, Wd1, bd1)
    return h @ Wd2 + bd2


# speedup vs baseline: 1.1674x; 1.1674x over previous
import functools

import jax
import jax.numpy as jnp
from jax import lax
from jax.experimental import pallas as pl
from jax.experimental.pallas import tpu as pltpu
from jax.experimental.pallas import tpu_sc as plsc

N = 50000
E = 800000
L = 100000
H = 64
HH = 32
F = 128

NT = 16
CH = 128

GB = 2
GR = GB * CH
IDEP = 4

EPAD = 802816
EPT = EPAD // NT
NCHE = EPT // CH
NGRP = NCHE // GB

NP = 51200
RPT = NP // NT
ZC = 200
RD = 3

DGB = 4
DGR = DGB * CH
LPAD = 106496
LPT = LPAD // NT
NCHL = LPT // CH
LGRP = NCHL // DGB

@functools.cache
def _mesh():
    return plsc.VectorSubcoreMesh(core_axis_name="c", subcore_axis_name="s",
                                  num_cores=2, num_subcores=NT)


def _sc_counts(seg2):
    @functools.partial(
        pl.kernel,
        out_type=jax.ShapeDtypeStruct((2, NP, 16), jnp.float32),
        mesh=_mesh(),
        compiler_params=pltpu.CompilerParams(use_tc_tiling_on_sc=False),
        scratch_types=[
            pltpu.VMEM_SHARED((NP, 16), jnp.float32),
            pltpu.VMEM((NCHE, CH), jnp.int32),
            pltpu.VMEM((CH, 16), jnp.float32),
            pltpu.VMEM((GR, 16), jnp.float32),
            pltpu.SemaphoreType.DMA,
        ],
    )
    def k(seg_h, out_h, acc, svb, ones_v, zbuf, ssem):
        c = lax.axis_index("c")
        s = lax.axis_index("s")
        z16 = jnp.zeros((16,), jnp.float32)
        o16 = jnp.ones((16,), jnp.float32)

        def fill(i, _):
            zbuf[i, :] = z16
            return _
        lax.fori_loop(0, GR, fill, None)

        def fill_o(i, _):
            ones_v[i, :] = o16
            return _
        lax.fori_loop(0, CH, fill_o, None)

        r0 = s * RPT

        def zero_acc(i, _):
            pltpu.sync_copy(zbuf, acc.at[pl.ds(r0 + i * GR, GR)])
            return _
        lax.fori_loop(0, RPT // GR, zero_acc, None)
        pltpu.sync_copy(seg_h.at[c].at[s], svb)
        plsc.subcore_barrier()

        ngr = NCHE // GB
        for b in range(GB):
            pltpu.async_copy(ones_v, acc.at[svb.at[b]], ssem, add=True)

        def body(g, _):
            for b in range(GB):
                pltpu.async_copy(ones_v, acc.at[svb.at[g * GB + b]], ssem,
                                 add=True)
            for b in range(GB):
                pltpu.make_async_copy(
                    ones_v, acc.at[svb.at[(g - 1) * GB + b]], ssem).wait()
            return _
        lax.fori_loop(1, ngr, body, None)
        for b in range(GB):
            pltpu.make_async_copy(
                ones_v, acc.at[svb.at[(ngr - 1) * GB + b]], ssem).wait()
        plsc.subcore_barrier()
        pltpu.sync_copy(acc.at[pl.ds(r0, RPT)], out_h.at[c].at[pl.ds(r0, RPT)])

    return k(seg2)


def _sc_agg(tab2, gidx2, sidx):
    @functools.partial(
        pl.kernel,
        out_type=jax.ShapeDtypeStruct((2, NP, HH), jnp.float32),
        mesh=_mesh(),
        compiler_params=pltpu.CompilerParams(use_tc_tiling_on_sc=False),
        scratch_types=[
            pltpu.VMEM_SHARED((NP, HH), jnp.float32),
            pltpu.VMEM((IDEP, GB, CH), jnp.int32),
            pltpu.VMEM((IDEP, GB, CH), jnp.int32),
            pltpu.VMEM((RD, GR, HH), jnp.float32),
            pltpu.SemaphoreType.DMA,
            pltpu.SemaphoreType.DMA,
            pltpu.SemaphoreType.DMA,
        ],
    )
    def k(tab_h, g_h, s_h, out_h, acc, gvb, svb, rows, isem, gsem, ssem):
        c = lax.axis_index("c")
        s = lax.axis_index("s")
        z16 = jnp.zeros((16,), jnp.float32)

        def fill_z(i, _):
            rows[0, i, pl.ds(0, 16)] = z16
            rows[0, i, pl.ds(16, 16)] = z16
            return _
        lax.fori_loop(0, GR, fill_z, None)

        r0 = s * RPT

        def zero_acc(i, _):
            pltpu.sync_copy(rows.at[0].at[pl.ds(0, ZC)],
                            acc.at[pl.ds(r0 + i * ZC, ZC)])
            return _
        lax.fori_loop(0, RPT // ZC, zero_acc, None)

        def idx_load(g, q):
            pltpu.async_copy(g_h.at[c].at[s].at[pl.ds(g * GB, GB)],
                             gvb.at[q], isem)
            pltpu.async_copy(s_h.at[s].at[pl.ds(g * GB, GB)],
                             svb.at[q], isem)

        def idx_drain(g, q):
            pltpu.make_async_copy(g_h.at[c].at[s].at[pl.ds(g * GB, GB)],
                                  gvb.at[q], isem).wait()
            pltpu.make_async_copy(s_h.at[s].at[pl.ds(g * GB, GB)],
                                  svb.at[q], isem).wait()

        def gath(q, p):
            for b in range(GB):
                pltpu.async_copy(tab_h.at[gvb.at[q].at[b]],
                                 rows.at[p].at[pl.ds(b * CH, CH)], gsem)

        def gath_drain(q, p):
            for b in range(GB):
                pltpu.make_async_copy(tab_h.at[gvb.at[q].at[b]],
                                      rows.at[p].at[pl.ds(b * CH, CH)],
                                      gsem).wait()

        def scat(q, p):
            for b in range(GB):
                pltpu.async_copy(rows.at[p].at[pl.ds(b * CH, CH)],
                                 acc.at[svb.at[q].at[b]], ssem, add=True)

        def scat_drain(q, p):
            for b in range(GB):
                pltpu.make_async_copy(rows.at[p].at[pl.ds(b * CH, CH)],
                                      acc.at[svb.at[q].at[b]], ssem).wait()

        idx_load(0, 0)
        idx_load(1, 1)
        idx_drain(0, 0)
        gath(0, 0)
        plsc.subcore_barrier()

        def body(g, _):
            p = g % RD
            q = g % IDEP

            @pl.when(g >= RD)
            def _free_rows():
                scat_drain((g - RD) % IDEP, p)

            @pl.when(g + 1 < NGRP)
            def _prefetch():
                idx_load(g + 1, (g + 1) % IDEP)

            idx_drain(g, q)
            gath(q, p)
            gath_drain((g - 1) % IDEP, (g - 1) % RD)
            scat((g - 1) % IDEP, (g - 1) % RD)
            return _
        lax.fori_loop(1, NGRP, body, None)

        last = NGRP - 1
        for gg in range(last - RD + 1, last):
            scat_drain(gg % IDEP, gg % RD)
        gath_drain(last % IDEP, last % RD)
        scat(last % IDEP, last % RD)
        scat_drain(last % IDEP, last % RD)

        plsc.subcore_barrier()
        pltpu.sync_copy(acc.at[pl.ds(r0, RPT)], out_h.at[c].at[pl.ds(r0, RPT)])

    return k(tab2, gidx2, sidx)


def _sc_dec_gather(uf, mf, eli2):
    @functools.partial(
        pl.kernel,
        out_type=jax.ShapeDtypeStruct((2, LPAD, H), jnp.float32),
        mesh=_mesh(),
        compiler_params=pltpu.CompilerParams(use_tc_tiling_on_sc=False),
        scratch_types=[
            pltpu.VMEM((NCHL, CH), jnp.int32),
            pltpu.VMEM((2, DGR, H), jnp.float32),
            pltpu.SemaphoreType.DMA,
            pltpu.SemaphoreType.DMA,
        ],
    )
    def k(uf_h, mf_h, e_h, out_h, evb, rows, gsem, osem):
        c = lax.axis_index("c")
        s = lax.axis_index("s")
        pltpu.sync_copy(e_h.at[c].at[s], evb)

        def gath(tab_h, g, p):
            for b in range(DGB):
                pltpu.async_copy(tab_h.at[evb.at[g * DGB + b]],
                                 rows.at[p].at[pl.ds(b * CH, CH)], gsem)

        def gath_drain(tab_h, g, p):
            for b in range(DGB):
                pltpu.make_async_copy(tab_h.at[evb.at[g * DGB + b]],
                                      rows.at[p].at[pl.ds(b * CH, CH)],
                                      gsem).wait()

        def out_ref(g):
            return out_h.at[c].at[pl.ds(s * LPT + g * DGR, DGR)]

        def run(tab_h):
            gath(tab_h, 0, 0)

            def body(g, _):
                p = g % 2
                @pl.when(g >= 2)
                def _drain():
                    pltpu.make_async_copy(rows.at[p], out_ref(g - 2),
                                          osem).wait()
                gath(tab_h, g, p)
                gath_drain(tab_h, g - 1, 1 - p)
                pltpu.async_copy(rows.at[1 - p], out_ref(g - 1), osem)
                return _
            lax.fori_loop(1, LGRP, body, None)
            last = LGRP - 1
            lp = last % 2
            pltpu.make_async_copy(rows.at[1 - lp], out_ref(last - 1),
                                  osem).wait()
            gath_drain(tab_h, last, lp)
            pltpu.sync_copy(rows.at[lp], out_ref(last))

        @pl.when(c == 0)
        def _u():
            run(uf_h)

        @pl.when(c == 1)
        def _m():
            run(mf_h)

    return k(uf, mf, eli2)


_BN = 2000


def _tc_prep(movie_x, user_emb, wp, bp):
    def body(mx_ref, ue_ref, wp_ref, bp_ref, om_ref, ou_ref):
        m = jnp.maximum(
            jnp.dot(mx_ref[...], wp_ref[...],
                    preferred_element_type=jnp.float32) + bp_ref[...], 0.0)
        om_ref[0] = m[:, :HH]
        om_ref[1] = m[:, HH:]
        ue = ue_ref[...]
        ou_ref[0] = ue[:, :HH]
        ou_ref[1] = ue[:, HH:]

    grid = (N // _BN,)
    return pl.pallas_call(
        body,
        grid=grid,
        in_specs=[
            pl.BlockSpec((_BN, F), lambda i: (i, 0)),
            pl.BlockSpec((_BN, H), lambda i: (i, 0)),
            pl.BlockSpec((F, H), lambda i: (0, 0)),
            pl.BlockSpec((1, H), lambda i: (0, 0)),
        ],
        out_specs=[
            pl.BlockSpec((2, _BN, HH), lambda i: (0, i, 0)),
            pl.BlockSpec((2, _BN, HH), lambda i: (0, i, 0)),
        ],
        out_shape=[
            jax.ShapeDtypeStruct((2, N, HH), jnp.float32),
            jax.ShapeDtypeStruct((2, N, HH), jnp.float32),
        ],
    )(movie_x, user_emb, wp, bp.reshape(1, H))


def _tc_combine1(s_agg, cnts, x_p, wl, bl, wr, which):
    def body(s_ref, c_ref, x_ref, wl_ref, bl_ref, wr_ref, o_ref):
        cnt = jnp.maximum(c_ref[0, :, :1], 1.0)
        wl_ = wl_ref[...]
        wr_ = wr_ref[...]
        a_lo = s_ref[0] / cnt
        a_hi = s_ref[1] / cnt
        z = (jnp.dot(a_lo, wl_[:HH], preferred_element_type=jnp.float32)
             + jnp.dot(a_hi, wl_[HH:], preferred_element_type=jnp.float32)
             + jnp.dot(x_ref[0], wr_[:HH], preferred_element_type=jnp.float32)
             + jnp.dot(x_ref[1], wr_[HH:], preferred_element_type=jnp.float32)
             + bl_ref[...])
        x_full = jnp.concatenate([x_ref[0], x_ref[1]], axis=1)
        r = x_full + jnp.maximum(z, 0.0)
        o_ref[0] = r[:, :HH]
        o_ref[1] = r[:, HH:]

    return pl.pallas_call(
        body,
        grid=(N // _BN,),
        in_specs=[
            pl.BlockSpec((2, _BN, HH), lambda i: (0, i, 0)),
            pl.BlockSpec((1, _BN, 16), lambda i: (which, i, 0)),
            pl.BlockSpec((2, _BN, HH), lambda i: (0, i, 0)),
            pl.BlockSpec((H, H), lambda i: (0, 0)),
            pl.BlockSpec((1, H), lambda i: (0, 0)),
            pl.BlockSpec((H, H), lambda i: (0, 0)),
        ],
        out_specs=pl.BlockSpec((2, _BN, HH), lambda i: (0, i, 0)),
        out_shape=jax.ShapeDtypeStruct((2, N, HH), jnp.float32),
    )(s_agg, cnts, x_p, wl, bl.reshape(1, H), wr)


def _tc_combine2(s_agg, cnts, x_p, wl, bl, wr, which):
    def body(s_ref, c_ref, x_ref, wl_ref, bl_ref, wr_ref, o_ref):
        cnt = jnp.maximum(c_ref[0, :, :1], 1.0)
        wl_ = wl_ref[...]
        wr_ = wr_ref[...]
        a_lo = s_ref[0] / cnt
        a_hi = s_ref[1] / cnt
        z = (jnp.dot(a_lo, wl_[:HH], preferred_element_type=jnp.float32)
             + jnp.dot(a_hi, wl_[HH:], preferred_element_type=jnp.float32)
             + jnp.dot(x_ref[0], wr_[:HH], preferred_element_type=jnp.float32)
             + jnp.dot(x_ref[1], wr_[HH:], preferred_element_type=jnp.float32)
             + bl_ref[...])
        nrm = jnp.sqrt(jnp.sum(z * z, axis=1, keepdims=True))
        o_ref[...] = z / jnp.maximum(nrm, 1e-12)

    return pl.pallas_call(
        body,
        grid=(N // _BN,),
        in_specs=[
            pl.BlockSpec((2, _BN, HH), lambda i: (0, i, 0)),
            pl.BlockSpec((1, _BN, 16), lambda i: (which, i, 0)),
            pl.BlockSpec((2, _BN, HH), lambda i: (0, i, 0)),
            pl.BlockSpec((H, H), lambda i: (0, 0)),
            pl.BlockSpec((1, H), lambda i: (0, 0)),
            pl.BlockSpec((H, H), lambda i: (0, 0)),
        ],
        out_specs=pl.BlockSpec((_BN, H), lambda i: (i, 0)),
        out_shape=jax.ShapeDtypeStruct((N, H), jnp.float32),
    )(s_agg, cnts, x_p, wl, bl.reshape(1, H), wr)


_BL = 2000


def _tc_decoder(g, wd1, bd1):
    def body(g_ref, w1_ref, b1_ref, o_ref):
        w1 = w1_ref[...]
        h = (jnp.dot(g_ref[0], w1[:H], preferred_element_type=jnp.float32)
             + jnp.dot(g_ref[1], w1[H:], preferred_element_type=jnp.float32)
             + b1_ref[...])
        o_ref[...] = jnp.maximum(h, 0.0)

    return pl.pallas_call(
        body,
        grid=(L // _BL,),
        in_specs=[
            pl.BlockSpec((2, _BL, H), lambda i: (0, i, 0)),
            pl.BlockSpec((2 * H, H), lambda i: (0, 0)),
            pl.BlockSpec((1, H), lambda i: (0, 0)),
        ],
        out_specs=pl.BlockSpec((_BL, H), lambda i: (i, 0)),
        out_shape=jax.ShapeDtypeStruct((L, H), jnp.float32),
    )(g, wd1, bd1.reshape(1, H))


def kernel(movie_x, edge_index, edge_label_index, user_emb, Wp, bp,
           W1m_l, b1m, W1m_r, W1u_l, b1u, W1u_r,
           W2m_l, b2m, W2m_r, W2u_l, b2u, W2u_r,
           Wd1, bd1, Wd2, bd2):
    i32 = jnp.int32
    src = edge_index[0].astype(i32)
    dst = edge_index[1].astype(i32)
    npad = EPAD - E
    pad0 = jnp.zeros((npad,), i32)
    padn = jnp.full((npad,), N, i32)
    src_g = jnp.concatenate([src, pad0])
    dst_g = jnp.concatenate([dst, pad0])
    src_s = jnp.concatenate([src, padn]).reshape(NT, NCHE, CH)
    dst_s = jnp.concatenate([dst, padn]).reshape(NT, NCHE, CH)
    srcg2 = jnp.concatenate([src_g, src_g + N]).reshape(2, NT, NCHE, CH)
    dstg2 = jnp.concatenate([dst_g, dst_g + N]).reshape(2, NT, NCHE, CH)
    seg2 = jnp.stack([dst_s, src_s])

    cnts = _sc_counts(seg2)
    movie0_p, user0_p = _tc_prep(movie_x, user_emb, Wp, bp)

    s1m = _sc_agg(user0_p.reshape(2 * N, HH), srcg2, dst_s)
    s1u = _sc_agg(movie0_p.reshape(2 * N, HH), dstg2, src_s)
    mr_p = _tc_combine1(s1m, cnts, movie0_p, W1m_l, b1m, W1m_r, which=0)
    ur_p = _tc_combine1(s1u, cnts, user0_p, W1u_l, b1u, W1u_r, which=1)

    s2m = _sc_agg(ur_p.reshape(2 * N, HH), srcg2, dst_s)
    s2u = _sc_agg(mr_p.reshape(2 * N, HH), dstg2, src_s)
    mf = _tc_combine2(s2m, cnts, mr_p, W2m_l, b2m, W2m_r, which=0)
    uf = _tc_combine2(s2u, cnts, ur_p, W2u_l, b2u, W2u_r, which=1)

    eli = edge_label_index.astype(i32)
    lpad0 = jnp.zeros((LPAD - L,), i32)
    e0 = jnp.concatenate([eli[0], lpad0])
    e1 = jnp.concatenate([eli[1], lpad0])
    eli2 = jnp.concatenate([e0, e1]).reshape(2, NT, NCHL, CH)

    g = _sc_dec_gather(uf, mf, eli2)
    h = _tc_decoder(g, Wd1, bd1)
    return h @ Wd2 + bd2
